# baseline (speedup 1.0000x reference)
import functools

import jax
import jax.numpy as jnp
from jax import lax
from jax.experimental import pallas as pl
from jax.experimental.pallas import tpu as pltpu
from jax.experimental.pallas import tpu_sc as plsc

DIM = 64
CHUNK = 512
NBUF = 3
PANEL = 512


def _tc_transpose_body(x1_ref, x2_ref, t1_ref, t2_ref, o_ref, *, last):
    i = pl.program_id(0)

    @pl.when(i < last)
    def _():
        o_ref[:, 0:DIM] = x1_ref[...].T
        o_ref[:, DIM:2 * DIM] = x2_ref[...].T

    @pl.when(i == last)
    def _():
        o_ref[:, 0:DIM] = t1_ref[...].T
        o_ref[:, DIM:2 * DIM] = t2_ref[...].T


def _gather_kernel(idx_hbm, table_hbm, out_hbm, idx_v, rows, g_sems, o_sems,
                   *, num_cores, b_per_w, n_chunks, n_b0):
    wid = lax.axis_index("s") * num_cores + lax.axis_index("c")
    base = wid * b_per_w

    pltpu.sync_copy(idx_hbm.at[wid], idx_v)

    def gather_start(g):
        s = g % NBUF
        return pltpu.async_copy(table_hbm.at[idx_v.at[g]], rows[s], g_sems[s])

    def write_start(g):
        s = g % NBUF
        c0 = base + g * CHUNK
        b1c = c0 // n_b0
        b0c = c0 % n_b0
        return pltpu.async_copy(
            rows[s], out_hbm.at[b1c, pl.ds(b0c, CHUNK)], o_sems[s])

    gathers = [None] * n_chunks
    writes = [None] * n_chunks
    gathers[0] = gather_start(0)
    for g in range(n_chunks):
        if g + 1 < n_chunks:
            if g + 1 >= NBUF:
                writes[g + 1 - NBUF].wait()
            gathers[g + 1] = gather_start(g + 1)
        gathers[g].wait()
        writes[g] = write_start(g)
    for g in range(max(0, n_chunks - NBUF), n_chunks):
        writes[g].wait()


def kernel(idxes, relEmbbed):
    b0, b1 = idxes.shape
    total = b0 * b1
    n_rows = relEmbbed.shape[0]
    info = plsc.get_sparse_core_info()
    num_workers = info.num_cores * info.num_subcores
    b_per_w = total // num_workers
    n_chunks = b_per_w // CHUNK

    n_blocks = -(-n_rows // (2 * PANEL))
    n_full_panels = (n_rows // PANEL)
    tail_start = (n_blocks - 1) * 2 * PANEL
    pad_rows = n_blocks * 2 * PANEL - n_rows
    tc_transpose = pl.pallas_call(
        functools.partial(_tc_transpose_body, last=n_blocks - 1),
        grid=(n_blocks,),
        in_specs=[
            pl.BlockSpec((DIM, PANEL),
                         lambda i: (0, jnp.minimum(2 * i, n_full_panels - 1))),
            pl.BlockSpec((DIM, PANEL),
                         lambda i: (0, jnp.minimum(2 * i + 1,
                                                   n_full_panels - 1))),
            pl.BlockSpec((DIM, PANEL), lambda i: (0, 0)),
            pl.BlockSpec((DIM, PANEL), lambda i: (0, 1)),
        ],
        out_specs=pl.BlockSpec((PANEL, 2 * DIM), lambda i: (i, 0)),
        out_shape=jax.ShapeDtypeStruct((n_blocks * PANEL, 2 * DIM),
                                       jnp.float32),
    )
    tt = relEmbbed.T
    tailp = jnp.pad(
        lax.slice(relEmbbed, (tail_start, 0), (n_rows, DIM)).T,
        ((0, 0), (0, pad_rows)))
    scratch = tc_transpose(tt, tt, tailp, tailp)
    tlin = scratch.reshape(n_blocks * PANEL * 2, DIM)

    r = idxes.T.astype(jnp.int32)
    q = jnp.bitwise_and(r, 1023)
    gidx = (jnp.bitwise_and(r, jnp.int32(~1023))
            + 2 * jnp.bitwise_and(q, 511) + (q >> 9))

    mesh = plsc.VectorSubcoreMesh(core_axis_name="c", subcore_axis_name="s")
    run = pl.kernel(
        functools.partial(
            _gather_kernel,
            num_cores=info.num_cores,
            b_per_w=b_per_w,
            n_chunks=n_chunks,
            n_b0=b0,
        ),
        mesh=mesh,
        compiler_params=pltpu.CompilerParams(use_tc_tiling_on_sc=False),
        out_type=jax.ShapeDtypeStruct((b1, b0, DIM), jnp.float32),
        scratch_types=[
            pltpu.VMEM((n_chunks, CHUNK), jnp.int32),
            [pltpu.VMEM((CHUNK, DIM), jnp.float32) for _ in range(NBUF)],
            [pltpu.SemaphoreType.DMA for _ in range(NBUF)],
            [pltpu.SemaphoreType.DMA for _ in range(NBUF)],
        ],
    )
    flat_idx = gidx.reshape(num_workers, n_chunks, CHUNK)
    out3 = run(flat_idx, tlin)
    return out3.transpose(1, 0, 2)


# speedup vs baseline: 1.8178x; 1.8178x over previous
import functools

import jax
import jax.numpy as jnp
from jax import lax
from jax.experimental import pallas as pl
from jax.experimental.pallas import tpu as pltpu
from jax.experimental.pallas import tpu_sc as plsc

DIM = 64
CHUNK = 512
NBUF = 3
PANEL = 8192


def _tc_transpose_body(x1_ref, x2_ref, t1_ref, t2_ref, eye_ref, o_ref, *,
                       last):
    i = pl.program_id(0)
    eye = eye_ref[...]

    def tr(x):
        return lax.dot_general(x, eye, (((0,), (0,)), ((), ())),
                               preferred_element_type=jnp.float32)

    @pl.when(i < last)
    def _():
        o_ref[:, 0:DIM] = tr(x1_ref[...])
        o_ref[:, DIM:2 * DIM] = tr(x2_ref[...])

    @pl.when(i == last)
    def _():
        o_ref[:, 0:DIM] = tr(t1_ref[...])
        o_ref[:, DIM:2 * DIM] = tr(t2_ref[...])


def _gather_kernel(idx_hbm, table_hbm, out_hbm, idx_v, rows, g_sems, o_sems,
                   *, num_cores, b_per_w, n_chunks, n_b0):
    wid = lax.axis_index("s") * num_cores + lax.axis_index("c")
    base = wid * b_per_w

    pltpu.sync_copy(idx_hbm.at[wid], idx_v)

    def gather_start(g):
        s = g % NBUF
        return pltpu.async_copy(table_hbm.at[idx_v.at[g]], rows[s], g_sems[s])

    def write_start(g):
        s = g % NBUF
        c0 = base + g * CHUNK
        b1c = c0 // n_b0
        b0c = c0 % n_b0
        return pltpu.async_copy(
            rows[s], out_hbm.at[b1c, pl.ds(b0c, CHUNK)], o_sems[s])

    gathers = [None] * n_chunks
    writes = [None] * n_chunks
    gathers[0] = gather_start(0)
    for g in range(n_chunks):
        if g + 1 < n_chunks:
            if g + 1 >= NBUF:
                writes[g + 1 - NBUF].wait()
            gathers[g + 1] = gather_start(g + 1)
        gathers[g].wait()
        writes[g] = write_start(g)
    for g in range(max(0, n_chunks - NBUF), n_chunks):
        writes[g].wait()


def kernel(idxes, relEmbbed):
    b0, b1 = idxes.shape
    total = b0 * b1
    n_rows = relEmbbed.shape[0]
    info = plsc.get_sparse_core_info()
    num_workers = info.num_cores * info.num_subcores
    b_per_w = total // num_workers
    n_chunks = b_per_w // CHUNK

    n_blocks = -(-n_rows // (2 * PANEL))
    n_full_panels = (n_rows // PANEL)
    tail_start = (n_blocks - 1) * 2 * PANEL
    pad_rows = n_blocks * 2 * PANEL - n_rows
    tc_transpose = pl.pallas_call(
        functools.partial(_tc_transpose_body, last=n_blocks - 1),
        grid=(n_blocks,),
        in_specs=[
            pl.BlockSpec((DIM, PANEL),
                         lambda i: (0, jnp.minimum(2 * i, n_full_panels - 1))),
            pl.BlockSpec((DIM, PANEL),
                         lambda i: (0, jnp.minimum(2 * i + 1,
                                                   n_full_panels - 1))),
            pl.BlockSpec((DIM, PANEL), lambda i: (0, 0)),
            pl.BlockSpec((DIM, PANEL), lambda i: (0, 1)),
            pl.BlockSpec((DIM, DIM), lambda i: (0, 0)),
        ],
        out_specs=pl.BlockSpec((PANEL, 2 * DIM), lambda i: (i, 0)),
        out_shape=jax.ShapeDtypeStruct((n_blocks * PANEL, 2 * DIM),
                                       jnp.float32),
    )
    tt = relEmbbed.T
    tailp = jnp.pad(
        lax.slice(relEmbbed, (tail_start, 0), (n_rows, DIM)).T,
        ((0, 0), (0, pad_rows)))
    scratch = tc_transpose(tt, tt, tailp, tailp,
                           jnp.eye(DIM, dtype=jnp.float32))
    tlin = scratch.reshape(n_blocks * PANEL * 2, DIM)

    shift = PANEL.bit_length() - 1
    r = idxes.T.astype(jnp.int32)
    q = jnp.bitwise_and(r, 2 * PANEL - 1)
    gidx = (jnp.bitwise_and(r, jnp.int32(~(2 * PANEL - 1)))
            + 2 * jnp.bitwise_and(q, PANEL - 1) + (q >> shift))

    mesh = plsc.VectorSubcoreMesh(core_axis_name="c", subcore_axis_name="s")
    run = pl.kernel(
        functools.partial(
            _gather_kernel,
            num_cores=info.num_cores,
            b_per_w=b_per_w,
            n_chunks=n_chunks,
            n_b0=b0,
        ),
        mesh=mesh,
        compiler_params=pltpu.CompilerParams(use_tc_tiling_on_sc=False),
        out_type=jax.ShapeDtypeStruct((b1, b0, DIM), jnp.float32),
        scratch_types=[
            pltpu.VMEM((n_chunks, CHUNK), jnp.int32),
            [pltpu.VMEM((CHUNK, DIM), jnp.float32) for _ in range(NBUF)],
            [pltpu.SemaphoreType.DMA for _ in range(NBUF)],
            [pltpu.SemaphoreType.DMA for _ in range(NBUF)],
        ],
    )
    flat_idx = gidx.reshape(num_workers, n_chunks, CHUNK)
    out3 = run(flat_idx, tlin)
    return out3.transpose(1, 0, 2)


# speedup vs baseline: 1.8215x; 1.0021x over previous
import functools

import jax
import jax.numpy as jnp
from jax import lax
from jax.experimental import pallas as pl
from jax.experimental.pallas import tpu as pltpu
from jax.experimental.pallas import tpu_sc as plsc

DIM = 64
CHUNK = 512
NBUF = 3
PANEL = 8192


def _tc_transpose_body(x1_ref, x2_ref, t1_ref, t2_ref, eye_ref, o_ref, *,
                       last):
    i = pl.program_id(0)
    del eye_ref

    @pl.when(i < last)
    def _():
        for k in range(PANEL // 128):
            sl = pl.ds(k * 128, 128)
            o_ref[sl, 0:DIM] = x1_ref[:, sl].T
            o_ref[sl, DIM:2 * DIM] = x2_ref[:, sl].T

    @pl.when(i == last)
    def _():
        for k in range(PANEL // 128):
            sl = pl.ds(k * 128, 128)
            o_ref[sl, 0:DIM] = t1_ref[:, sl].T
            o_ref[sl, DIM:2 * DIM] = t2_ref[:, sl].T


def _gather_kernel(idx_hbm, table_hbm, out_hbm, idx_v, rows, g_sems, o_sems,
                   *, num_cores, b_per_w, n_chunks, n_b0):
    wid = lax.axis_index("s") * num_cores + lax.axis_index("c")
    base = wid * b_per_w

    pltpu.sync_copy(idx_hbm.at[wid], idx_v)

    def gather_start(g):
        s = g % NBUF
        return pltpu.async_copy(table_hbm.at[idx_v.at[g]], rows[s], g_sems[s])

    def write_start(g):
        s = g % NBUF
        c0 = base + g * CHUNK
        b1c = c0 // n_b0
        b0c = c0 % n_b0
        return pltpu.async_copy(
            rows[s], out_hbm.at[b1c, pl.ds(b0c, CHUNK)], o_sems[s])

    gathers = [None] * n_chunks
    writes = [None] * n_chunks
    gathers[0] = gather_start(0)
    for g in range(n_chunks):
        if g + 1 < n_chunks:
            if g + 1 >= NBUF:
                writes[g + 1 - NBUF].wait()
            gathers[g + 1] = gather_start(g + 1)
        gathers[g].wait()
        writes[g] = write_start(g)
    for g in range(max(0, n_chunks - NBUF), n_chunks):
        writes[g].wait()


def kernel(idxes, relEmbbed):
    b0, b1 = idxes.shape
    total = b0 * b1
    n_rows = relEmbbed.shape[0]
    info = plsc.get_sparse_core_info()
    num_workers = info.num_cores * info.num_subcores
    b_per_w = total // num_workers
    n_chunks = b_per_w // CHUNK

    n_blocks = -(-n_rows // (2 * PANEL))
    n_full_panels = (n_rows // PANEL)
    tail_start = (n_blocks - 1) * 2 * PANEL
    pad_rows = n_blocks * 2 * PANEL - n_rows
    tc_transpose = pl.pallas_call(
        functools.partial(_tc_transpose_body, last=n_blocks - 1),
        grid=(n_blocks,),
        in_specs=[
            pl.BlockSpec((DIM, PANEL),
                         lambda i: (0, jnp.minimum(2 * i, n_full_panels - 1))),
            pl.BlockSpec((DIM, PANEL),
                         lambda i: (0, jnp.minimum(2 * i + 1,
                                                   n_full_panels - 1))),
            pl.BlockSpec((DIM, PANEL), lambda i: (0, 0)),
            pl.BlockSpec((DIM, PANEL), lambda i: (0, 1)),
            pl.BlockSpec((DIM, DIM), lambda i: (0, 0)),
        ],
        out_specs=pl.BlockSpec((PANEL, 2 * DIM), lambda i: (i, 0)),
        out_shape=jax.ShapeDtypeStruct((n_blocks * PANEL, 2 * DIM),
                                       jnp.float32),
    )
    tt = relEmbbed.T
    tailp = jnp.pad(
        lax.slice(relEmbbed, (tail_start, 0), (n_rows, DIM)).T,
        ((0, 0), (0, pad_rows)))
    scratch = tc_transpose(tt, tt, tailp, tailp,
                           jnp.eye(DIM, dtype=jnp.float32))
    tlin = scratch.reshape(n_blocks * PANEL * 2, DIM)

    shift = PANEL.bit_length() - 1
    r = idxes.T.astype(jnp.int32)
    q = jnp.bitwise_and(r, 2 * PANEL - 1)
    gidx = (jnp.bitwise_and(r, jnp.int32(~(2 * PANEL - 1)))
            + 2 * jnp.bitwise_and(q, PANEL - 1) + (q >> shift))

    mesh = plsc.VectorSubcoreMesh(core_axis_name="c", subcore_axis_name="s")
    run = pl.kernel(
        functools.partial(
            _gather_kernel,
            num_cores=info.num_cores,
            b_per_w=b_per_w,
            n_chunks=n_chunks,
            n_b0=b0,
        ),
        mesh=mesh,
        compiler_params=pltpu.CompilerParams(use_tc_tiling_on_sc=False),
        out_type=jax.ShapeDtypeStruct((b1, b0, DIM), jnp.float32),
        scratch_types=[
            pltpu.VMEM((n_chunks, CHUNK), jnp.int32),
            [pltpu.VMEM((CHUNK, DIM), jnp.float32) for _ in range(NBUF)],
            [pltpu.SemaphoreType.DMA for _ in range(NBUF)],
            [pltpu.SemaphoreType.DMA for _ in range(NBUF)],
        ],
    )
    flat_idx = gidx.reshape(num_workers, n_chunks, CHUNK)
    out3 = run(flat_idx, tlin)
    return out3.transpose(1, 0, 2)
